# Initial kernel scaffold; baseline (speedup 1.0000x reference)
import jax
import jax.numpy as jnp
from jax.experimental import pallas as pl


def kernel(x, tables):
    raise NotImplementedError("write your pallas kernel here")


# speedup vs baseline: 9.5634x; 9.5634x over previous
import jax
import jax.numpy as jnp
import numpy as np
from jax import lax
from jax.experimental import pallas as pl
from jax.experimental.pallas import tpu as pltpu
from jax.experimental.pallas import tpu_sc as plsc

_FIELD_DIMS = [4000] * 26
_F = 26
_V = sum(_FIELD_DIMS)
_D = 16
_B = 4096
_OFFS = np.array((0, *np.cumsum(_FIELD_DIMS)[:-1]), dtype=np.int32)

_NC, _NS = 2, 16
_NW = _NC * _NS
_RPW = _B // _NW
_R = 2
_NG = _RPW // _R
_FP = 32
_SLOTS = _F * _FP
_NIDX = _R * _SLOTS
_CH = 128
_NCH = _NIDX // _CH

_PAIRS = [(i, j) for i in range(_F - 1) for j in range(i + 1, _F)]
_NP = len(_PAIRS)


def _ffm_body(xi_hbm, tab_hbm, out_hbm, xi_v, idx_v, rows_v, out_v, sem):
    wid = lax.axis_index("s") * _NC + lax.axis_index("c")
    row_base = wid * _RPW

    def group(g, carry):
        row0 = row_base + g * _R
        pltpu.sync_copy(xi_hbm.at[pl.ds(row0, _R)], xi_v)
        for r in range(_R):
            half = [xi_v[r, pl.ds(0, 16)], xi_v[r, pl.ds(16, 16)]]
            for t in range(_F):
                for h in range(2):
                    n0 = r * _SLOTS + t * _FP + h * 16
                    idx_v[n0 // _CH, pl.ds(n0 % _CH, 16)] = half[h] + t * _V
        copies = [
            pltpu.async_copy(tab_hbm.at[idx_v.at[c]], rows_v.at[c], sem)
            for c in range(_NCH)
        ]
        for cp in copies:
            cp.wait()
        for r in range(_R):
            for p, (i, j) in enumerate(_PAIRS):
                na = r * _SLOTS + j * _FP + i
                nb = r * _SLOTS + i * _FP + j
                out_v[r, p] = rows_v[na // _CH, na % _CH] * rows_v[nb // _CH, nb % _CH]
        pltpu.sync_copy(out_v, out_hbm.at[pl.ds(row0, _R)])
        return carry

    lax.fori_loop(0, _NG, group, 0)


def kernel(x, tables):
    xi = x.astype(jnp.int32) + jnp.asarray(_OFFS)
    xi_pad = jnp.pad(xi, ((0, 0), (0, _FP - _F)))
    tab_flat = tables.reshape(_F * _V, _D)
    run = pl.kernel(
        _ffm_body,
        mesh=plsc.VectorSubcoreMesh(core_axis_name="c", subcore_axis_name="s"),
        out_type=jax.ShapeDtypeStruct((_B, _NP, _D), jnp.float32),
        scratch_types=[
            pltpu.VMEM((_R, _FP), jnp.int32),
            pltpu.VMEM((_NCH, _CH), jnp.int32),
            pltpu.VMEM((_NCH, _CH, _D), jnp.float32),
            pltpu.VMEM((_R, _NP, _D), jnp.float32),
            pltpu.SemaphoreType.DMA,
        ],
        compiler_params=pltpu.CompilerParams(use_tc_tiling_on_sc=False),
    )
    return run(xi_pad, tab_flat)


# speedup vs baseline: 9.8414x; 1.0291x over previous
import jax
import jax.numpy as jnp
import numpy as np
from jax import lax
from jax.experimental import pallas as pl
from jax.experimental.pallas import tpu as pltpu
from jax.experimental.pallas import tpu_sc as plsc

_FIELD_DIMS = [4000] * 26
_F = 26
_V = sum(_FIELD_DIMS)
_D = 16
_B = 4096
_OFFS = np.array((0, *np.cumsum(_FIELD_DIMS)[:-1]), dtype=np.int32)

_NC, _NS = 2, 16
_NW = _NC * _NS
_RPW = _B // _NW
_R = 2
_NG = _RPW // _R
_FP = 32
_SLOTS = _F * _FP
_NIDX = _R * _SLOTS
_CH = 128
_NCH = _NIDX // _CH

_PAIRS = [(i, j) for i in range(_F - 1) for j in range(i + 1, _F)]
_NP = len(_PAIRS)


def _ffm_body(xi_hbm, tab_hbm, out_hbm, xi_v, idx_v, rows_v, out_v,
              gsem0, gsem1, osem0, osem1):
    wid = lax.axis_index("s") * _NC + lax.axis_index("c")
    row_base = wid * _RPW
    pltpu.sync_copy(xi_hbm.at[pl.ds(row_base, _RPW)], xi_v)
    gsems = (gsem0, gsem1)
    osems = (osem0, osem1)

    def fire_gather(g, buf):
        for r in range(_R):
            row = g * _R + r
            half = [xi_v[row, pl.ds(0, 16)], xi_v[row, pl.ds(16, 16)]]
            for t in range(_F):
                for h in range(2):
                    n0 = r * _SLOTS + t * _FP + h * 16
                    idx_v[buf, n0 // _CH, pl.ds(n0 % _CH, 16)] = half[h] + t * _V
        for c in range(_NCH):
            pltpu.async_copy(tab_hbm.at[idx_v.at[buf, c]], rows_v.at[buf, c],
                             gsems[buf])

    def wait_gather(buf):
        for c in range(_NCH):
            pltpu.make_async_copy(tab_hbm.at[idx_v.at[buf, c]],
                                  rows_v.at[buf, c], gsems[buf]).wait()

    def compute_and_write(g, buf):
        for r in range(_R):
            for p, (i, j) in enumerate(_PAIRS):
                na = r * _SLOTS + j * _FP + i
                nb = r * _SLOTS + i * _FP + j
                out_v[buf, r, p] = (rows_v[buf, na // _CH, na % _CH]
                                    * rows_v[buf, nb // _CH, nb % _CH])
        row0 = row_base + g * _R
        pltpu.async_copy(out_v.at[buf], out_hbm.at[pl.ds(row0, _R)], osems[buf])

    def wait_write(buf):
        pltpu.make_async_copy(out_v.at[buf], out_hbm.at[pl.ds(row_base, _R)],
                              osems[buf]).wait()

    fire_gather(0, 0)

    def body(k, carry):
        g0 = k * 2
        fire_gather(g0 + 1, 1)

        @pl.when(k > 0)
        def _():
            wait_write(0)
            wait_write(1)

        wait_gather(0)
        compute_and_write(g0, 0)
        fire_gather(jnp.minimum(g0 + 2, _NG - 1), 0)
        wait_gather(1)
        compute_and_write(g0 + 1, 1)
        return carry

    lax.fori_loop(0, _NG // 2, body, 0)
    wait_gather(0)
    wait_write(0)
    wait_write(1)


def kernel(x, tables):
    xi = x.astype(jnp.int32) + jnp.asarray(_OFFS)
    xi_pad = jnp.pad(xi, ((0, 0), (0, _FP - _F)))
    tab_flat = tables.reshape(_F * _V, _D)
    run = pl.kernel(
        _ffm_body,
        mesh=plsc.VectorSubcoreMesh(core_axis_name="c", subcore_axis_name="s"),
        out_type=jax.ShapeDtypeStruct((_B, _NP, _D), jnp.float32),
        scratch_types=[
            pltpu.VMEM((_RPW, _FP), jnp.int32),
            pltpu.VMEM((2, _NCH, _CH), jnp.int32),
            pltpu.VMEM((2, _NCH, _CH, _D), jnp.float32),
            pltpu.VMEM((2, _R, _NP, _D), jnp.float32),
            pltpu.SemaphoreType.DMA,
            pltpu.SemaphoreType.DMA,
            pltpu.SemaphoreType.DMA,
            pltpu.SemaphoreType.DMA,
        ],
        compiler_params=pltpu.CompilerParams(use_tc_tiling_on_sc=False),
    )
    return run(xi_pad, tab_flat)
